# Initial kernel scaffold; baseline (speedup 1.0000x reference)
import jax
import jax.numpy as jnp
from jax.experimental import pallas as pl


def kernel(X, I):
    raise NotImplementedError("write your pallas kernel here")


# speedup vs baseline: 28.9367x; 28.9367x over previous
import functools

import jax
import jax.numpy as jnp
from jax import lax
from jax.experimental import pallas as pl
from jax.experimental.pallas import tpu as pltpu
from jax.experimental.pallas import tpu_sc as plsc

DIM = 64
K = 16
NB = 64
B = 8
N = 65536

CH_TC = 2048
CH_SC = 8192
ROWS = B * K


def _tc_body(x_ref, iw_ref, o_ref):
    x = x_ref[0]
    iw = iw_ref[...]
    s = lax.dot_general(iw, x, (((1,), (1,)), ((), ())),
                        preferred_element_type=jnp.float32)
    o_ref[...] = jax.nn.sigmoid(s) * jnp.float32(NB) + jnp.float32(0.5)


def _tc_stage(X, Iw):
    return pl.pallas_call(
        _tc_body,
        grid=(B, N // CH_TC),
        in_specs=[
            pl.BlockSpec((1, CH_TC, DIM), lambda b, n: (b, n, 0)),
            pl.BlockSpec((K, DIM), lambda b, n: (0, 0)),
        ],
        out_specs=pl.BlockSpec((K, CH_TC), lambda b, n: (b, n)),
        out_shape=jax.ShapeDtypeStruct((ROWS, N), jnp.float32),
        compiler_params=pltpu.CompilerParams(
            dimension_semantics=("parallel", "parallel")),
    )(X, Iw)


def _make_sc_stage():
    info = plsc.get_sparse_core_info()
    nc, ns = info.num_cores, info.num_subcores
    nw = nc * ns
    rows_per_w = ROWS // nw
    nchunk = N // CH_SC
    mesh = plsc.VectorSubcoreMesh(core_axis_name="c", subcore_axis_name="s")

    @functools.partial(
        pl.kernel,
        mesh=mesh,
        out_type=jax.ShapeDtypeStruct((ROWS * NB,), jnp.float32),
        scratch_types=[
            pltpu.VMEM((CH_SC,), jnp.float32),
            pltpu.VMEM((16 * NB,), jnp.float32),
            pltpu.VMEM((rows_per_w * NB,), jnp.float32),
        ],
        compiler_params=pltpu.CompilerParams(needs_layout_passes=False),
    )
    def sc_kernel(pos_hbm, out_hbm, buf, hist, res):
        wid = lax.axis_index("s") * nc + lax.axis_index("c")
        row0 = wid * rows_per_w
        lanebase = lax.iota(jnp.int32, 16) * NB
        zeros = jnp.zeros((16,), jnp.float32)

        def vreg_body(j, carry):
            off = pl.multiple_of(j * 16, 16)
            pv = buf[pl.ds(off, 16)]
            iq = pv.astype(jnp.int32)
            fr = pv - iq.astype(jnp.float32)
            i0c = jnp.maximum(iq - 1, 0)
            i1c = jnp.minimum(iq, NB - 1)
            plsc.addupdate_scatter(hist, [lanebase + i0c],
                                   jnp.float32(1.0) - fr)
            plsc.addupdate_scatter(hist, [lanebase + i1c], fr)
            return carry

        for p in range(rows_per_w):
            row = row0 + p
            for z in range(NB):
                hist[pl.ds(z * 16, 16)] = zeros
            for c in range(nchunk):
                pltpu.sync_copy(pos_hbm.at[row, pl.ds(c * CH_SC, CH_SC)], buf)
                lax.fori_loop(0, CH_SC // 16, vreg_body, 0)
            for g in range(NB // 16):
                acc = zeros
                for l in range(16):
                    acc = acc + hist[pl.ds(l * NB + g * 16, 16)]
                res[pl.ds(p * NB + g * 16, 16)] = acc * jnp.float32(1.0 / N)
        pltpu.sync_copy(res, out_hbm.at[pl.ds(row0 * NB, rows_per_w * NB)])

    return sc_kernel


def kernel(X, I):
    posq = _tc_stage(X, I[0])
    hist = _make_sc_stage()(posq)
    return hist.reshape(B, K * NB)


# speedup vs baseline: 30.7460x; 1.0625x over previous
import functools

import jax
import jax.numpy as jnp
from jax import lax
from jax.experimental import pallas as pl
from jax.experimental.pallas import tpu as pltpu
from jax.experimental.pallas import tpu_sc as plsc

DIM = 64
K = 16
NB = 64
B = 8
N = 65536

CH_TC = 2048
CH_SC = 8192
ROWS = B * K


def _tc_body(x_ref, iw_ref, o_ref):
    x = x_ref[0]
    iw = iw_ref[...]
    s = lax.dot_general(iw, x, (((1,), (1,)), ((), ())),
                        preferred_element_type=jnp.float32)
    o_ref[...] = jax.nn.sigmoid(s) * jnp.float32(NB) + jnp.float32(0.5)


def _tc_stage(X, Iw):
    return pl.pallas_call(
        _tc_body,
        grid=(B, N // CH_TC),
        in_specs=[
            pl.BlockSpec((1, CH_TC, DIM), lambda b, n: (b, n, 0)),
            pl.BlockSpec((K, DIM), lambda b, n: (0, 0)),
        ],
        out_specs=pl.BlockSpec((K, CH_TC), lambda b, n: (b, n)),
        out_shape=jax.ShapeDtypeStruct((ROWS, N), jnp.float32),
        compiler_params=pltpu.CompilerParams(
            dimension_semantics=("parallel", "parallel")),
    )(X, Iw)


def _make_sc_stage():
    info = plsc.get_sparse_core_info()
    nc, ns = info.num_cores, info.num_subcores
    nw = nc * ns
    rows_per_w = ROWS // nw
    nchunk = N // CH_SC
    mesh = plsc.VectorSubcoreMesh(core_axis_name="c", subcore_axis_name="s")

    @functools.partial(
        pl.kernel,
        mesh=mesh,
        out_type=jax.ShapeDtypeStruct((ROWS * NB,), jnp.float32),
        scratch_types=[
            pltpu.VMEM((CH_SC,), jnp.float32),
            pltpu.VMEM((CH_SC,), jnp.float32),
            pltpu.VMEM((16 * NB,), jnp.float32),
            pltpu.VMEM((rows_per_w * NB,), jnp.float32),
            pltpu.SemaphoreType.DMA,
            pltpu.SemaphoreType.DMA,
        ],
        compiler_params=pltpu.CompilerParams(needs_layout_passes=False),
    )
    def sc_kernel(pos_hbm, out_hbm, buf0, buf1, hist, res, sem0, sem1):
        wid = lax.axis_index("s") * nc + lax.axis_index("c")
        row0 = wid * rows_per_w
        lanebase = lax.iota(jnp.int32, 16) * NB
        zeros = jnp.zeros((16,), jnp.float32)
        bufs = (buf0, buf1)
        sems = (sem0, sem1)

        def make_body(buf):
            def vreg_body(j, carry):
                off = pl.multiple_of(j * 16, 16)
                pv = buf[pl.ds(off, 16)]
                iq = pv.astype(jnp.int32)
                fr = pv - iq.astype(jnp.float32)
                i0c = jnp.maximum(iq - 1, 0)
                i1c = jnp.minimum(iq, NB - 1)
                plsc.addupdate_scatter(hist, [lanebase + i0c],
                                       jnp.float32(1.0) - fr)
                plsc.addupdate_scatter(hist, [lanebase + i1c], fr)
                return carry
            return vreg_body

        steps = [(p, c) for p in range(rows_per_w) for c in range(nchunk)]

        def start(i):
            p, c = steps[i]
            return pltpu.async_copy(
                pos_hbm.at[row0 + p, pl.ds(c * CH_SC, CH_SC)],
                bufs[i % 2], sems[i % 2])

        pending = start(0)
        for i, (p, c) in enumerate(steps):
            if c == 0:
                for z in range(NB):
                    hist[pl.ds(z * 16, 16)] = zeros
            pending.wait()
            if i + 1 < len(steps):
                pending = start(i + 1)
            lax.fori_loop(0, CH_SC // 16, make_body(bufs[i % 2]), 0,
                          unroll=8)
            if c == nchunk - 1:
                for g in range(NB // 16):
                    acc = zeros
                    for l in range(16):
                        acc = acc + hist[pl.ds(l * NB + g * 16, 16)]
                    res[pl.ds(p * NB + g * 16, 16)] = acc * jnp.float32(1.0 / N)
        pltpu.sync_copy(res, out_hbm.at[pl.ds(row0 * NB, rows_per_w * NB)])

    return sc_kernel


def kernel(X, I):
    posq = _tc_stage(X, I[0])
    hist = _make_sc_stage()(posq)
    return hist.reshape(B, K * NB)
